# Initial kernel scaffold; baseline (speedup 1.0000x reference)
import jax
import jax.numpy as jnp
from jax.experimental import pallas as pl


def kernel(x, edge_index, W, b, Wl, bl):
    raise NotImplementedError("write your pallas kernel here")


# speedup vs baseline: 1.0131x; 1.0131x over previous
import functools

import jax
import jax.numpy as jnp
from jax.experimental import pallas as pl
from jax.experimental.pallas import tpu as pltpu

N_NODES = 10000
N_EDGES = 320000
D = 128
EPS = 4e-3
MIN_NORM = 1e-15


def _artanh(x):
    x = jnp.clip(x, -1.0 + 1e-7, 1.0 - 1e-7)
    return 0.5 * jnp.log((1.0 + x) / (1.0 - x))


def _rownorm(x):
    return jnp.clip(jnp.sqrt(jnp.sum(x * x, axis=-1, keepdims=True)), MIN_NORM, None)


def _proj(x):
    n = _rownorm(x)
    maxnorm = 1.0 - EPS
    return jnp.where(n > maxnorm, x / n * maxnorm, x)


def _xtan_body(x_ref, w_ref, b_ref, out_ref):
    x = x_ref[...]
    w = w_ref[...]
    b = b_ref[...]
    x_norm = _rownorm(x)
    mx = jax.lax.dot_general(x, w, (((1,), (1,)), ((), ())),
                             preferred_element_type=jnp.float32)
    mx_norm = _rownorm(mx)
    res_c = jnp.tanh(mx_norm / x_norm * _artanh(x_norm)) * mx / mx_norm
    cond = jnp.all(mx == 0.0, axis=-1, keepdims=True)
    res = jnp.where(cond, 0.0, res_c)
    res = _proj(res)
    b_norm = _rownorm(b)
    hb = _proj(jnp.tanh(b_norm) * b / b_norm)
    x2 = jnp.sum(res * res, axis=-1, keepdims=True)
    y2 = jnp.sum(hb * hb, axis=-1, keepdims=True)
    xy = jnp.sum(res * hb, axis=-1, keepdims=True)
    num = (1.0 + 2.0 * xy + y2) * res + (1.0 - x2) * hb
    den = 1.0 + 2.0 * xy + x2 * y2
    h = _proj(num / jnp.clip(den, MIN_NORM, None))
    h_norm = _rownorm(h)
    out_ref[...] = _artanh(h_norm) * h / h_norm


def _compute_x_tan(x, W, b):
    return pl.pallas_call(
        _xtan_body,
        out_shape=jax.ShapeDtypeStruct((N_NODES, D), jnp.float32),
    )(x, W, b.reshape(1, D))


def _final_body(xt_ref, a_ref, out_ref):
    out = xt_ref[...] + jnp.maximum(a_ref[...], 0.0)
    n = _rownorm(out)
    out = _proj(jnp.tanh(n) * out / n)
    out = jnp.maximum(out, 0.0)
    n = _rownorm(out)
    out_ref[...] = _proj(jnp.tanh(n) * out / n)


def _final_stage(x_tan, a_x_raw):
    return pl.pallas_call(
        _final_body,
        out_shape=jax.ShapeDtypeStruct((N_NODES, D), jnp.float32),
    )(x_tan, a_x_raw)


def kernel(x, edge_index, W, b, Wl, bl):
    x_tan = _compute_x_tan(x, W, b)
    src = edge_index[0]
    dst = edge_index[1]
    n = N_NODES

    def aggregate(v):
        return jax.ops.segment_sum(v[src], dst, num_segments=n)

    sum_neigh = aggregate(x_tan)
    ew = (src != dst).astype(jnp.float32)
    deg = jax.ops.segment_sum(ew, src, num_segments=n)
    dinv = jnp.where(deg > 0, deg ** -0.5, 0.0)
    norm_e = -dinv[src] * ew * dinv[dst]
    info = x_tan + jax.ops.segment_sum(norm_e[:, None] * x_tan[src], dst,
                                       num_segments=n)
    score = jnp.sum(jnp.abs(info), axis=1)
    k = int(n * 0.75)
    values = jax.lax.top_k(score, k)[0]
    T = jnp.min(values)
    sel = jnp.where(score > T, 1.0, 0.0)[:, None]
    sum_sel = aggregate(sel * x_tan)
    concat = jnp.concatenate([sum_sel, sum_neigh], axis=-1)
    w_sel = jax.nn.sigmoid(concat @ Wl.T + bl)
    a_x_raw = aggregate(w_sel * sel * x_tan)
    return _final_stage(x_tan, a_x_raw)


# speedup vs baseline: 8.9984x; 8.8821x over previous
import functools

import jax
import jax.numpy as jnp
from jax import lax
from jax.experimental import pallas as pl
from jax.experimental.pallas import tpu as pltpu
from jax.experimental.pallas import tpu_sc as plsc

N_NODES = 10000
N_EDGES = 320000
D = 128
EPS = 4e-3
MIN_NORM = 1e-15
TOPK = int(N_NODES * 0.75)

NC, NS = 2, 16
E_PER_T = N_EDGES // NS
CH = 80
NCH = E_PER_T // CH
NR = 5120
ACC_R = 5128
STRIPE = NR // NS
ZR = 16
HN = 10240
HIST_PER_TILE = HN // NS


def _artanh(x):
    x = jnp.clip(x, -1.0 + 1e-7, 1.0 - 1e-7)
    return 0.5 * jnp.log((1.0 + x) / (1.0 - x))


def _rownorm(x):
    return jnp.clip(jnp.sqrt(jnp.sum(x * x, axis=-1, keepdims=True)),
                    MIN_NORM, None)


def _proj(x):
    n = _rownorm(x)
    maxnorm = 1.0 - EPS
    return jnp.where(n > maxnorm, x / n * maxnorm, x)


def _expmap0_proj(x):
    n = _rownorm(x)
    return _proj(jnp.tanh(n) * x / n)


def _acc_full(accp_ref):
    return jnp.concatenate(
        [accp_ref[0], accp_ref[1, :N_NODES - NR]], axis=0)


def _tc1_body(x_ref, w_ref, b_ref, out_ref):
    x = x_ref[...]
    w = w_ref[...]
    b = b_ref[...]
    x_norm = _rownorm(x)
    mx = lax.dot_general(x, w, (((1,), (1,)), ((), ())),
                         preferred_element_type=jnp.float32)
    mx_norm = _rownorm(mx)
    res_c = jnp.tanh(mx_norm / x_norm * _artanh(x_norm)) * mx / mx_norm
    cond = jnp.all(mx == 0.0, axis=-1, keepdims=True)
    res = _proj(jnp.where(cond, 0.0, res_c))
    b_norm = _rownorm(b)
    hb = _proj(jnp.tanh(b_norm) * b / b_norm)
    x2 = jnp.sum(res * res, axis=-1, keepdims=True)
    y2 = jnp.sum(hb * hb, axis=-1, keepdims=True)
    xy = jnp.sum(res * hb, axis=-1, keepdims=True)
    num = (1.0 + 2.0 * xy + y2) * res + (1.0 - x2) * hb
    den = 1.0 + 2.0 * xy + x2 * y2
    h = _proj(num / jnp.clip(den, MIN_NORM, None))
    h_norm = _rownorm(h)
    out_ref[...] = _artanh(h_norm) * h / h_norm


def _tc1(x, W, b):
    return pl.pallas_call(
        _tc1_body,
        out_shape=jax.ShapeDtypeStruct((N_NODES, D), jnp.float32),
    )(x, W, b.reshape(1, D))


def _tc2_body(deg_ref, ns_ref, xt_ref, z_ref, dinv_ref, nso_ref):
    deg = deg_ref[...]
    dinv = jnp.where(deg > 0.0, lax.rsqrt(jnp.maximum(deg, MIN_NORM)), 0.0)
    dinv_ref[...] = dinv
    nso_ref[...] = ns_ref[...]
    z_ref[...] = dinv * xt_ref[...]


def _tc2(deg, ns, x_tan):
    return pl.pallas_call(
        _tc2_body,
        out_shape=(
            jax.ShapeDtypeStruct((N_NODES, D), jnp.float32),
            jax.ShapeDtypeStruct((N_NODES, 1), jnp.float32),
            jax.ShapeDtypeStruct((N_NODES, 1), jnp.float32),
        ),
    )(deg, ns, x_tan)


def _tc3_body(xt_ref, z_ref, dinv_ref, ns_ref, accz_ref, y2_ref, sel_ref):
    xt = xt_ref[...]
    accz = _acc_full(accz_ref)
    info = xt - dinv_ref[...] * (accz - ns_ref[...] * z_ref[...])
    score = jnp.sum(jnp.abs(info), axis=-1, keepdims=True)
    bits = lax.bitcast_convert_type(score, jnp.int32)

    def body(_, carry):
        lo, hi = carry
        mid = lo + (hi - lo) // 2
        cnt = jnp.sum((bits >= mid).astype(jnp.int32))
        big = cnt >= TOPK
        return jnp.where(big, mid, lo), jnp.where(big, hi, mid)

    lo, _ = lax.fori_loop(0, 31, body,
                          (jnp.int32(0), jnp.int32(0x7F800000)))
    sel = (bits > lo).astype(jnp.float32)
    sel_ref[...] = sel
    y2_ref[...] = sel * xt


def _tc3(x_tan, z, dinv, nself, acczp):
    return pl.pallas_call(
        _tc3_body,
        out_shape=(
            jax.ShapeDtypeStruct((N_NODES, D), jnp.float32),
            jax.ShapeDtypeStruct((N_NODES, 1), jnp.float32),
        ),
    )(x_tan, z, dinv, nself, acczp)


def _tc4_body(accx_ref, accs_ref, xt_ref, sel_ref, wl1_ref, wl2_ref, bl_ref,
              y3_ref):
    sum_neigh = _acc_full(accx_ref)
    sum_sel = _acc_full(accs_ref)
    logit = (jnp.sum(sum_sel * wl1_ref[...], axis=-1, keepdims=True)
             + jnp.sum(sum_neigh * wl2_ref[...], axis=-1, keepdims=True)
             + bl_ref[0, 0])
    w = jax.nn.sigmoid(logit) * sel_ref[...]
    y3_ref[...] = w * xt_ref[...]


def _tc4(accxp, accsp, x_tan, sel, Wl, bl):
    return pl.pallas_call(
        _tc4_body,
        out_shape=jax.ShapeDtypeStruct((N_NODES, D), jnp.float32),
    )(accxp, accsp, x_tan, sel, Wl[:, :D], Wl[:, D:], bl.reshape(1, 1))


def _tc5_body(xt_ref, accy_ref, out_ref):
    a_x = jnp.maximum(_acc_full(accy_ref), 0.0)
    out = xt_ref[...] + a_x
    out = _expmap0_proj(out)
    out = jnp.maximum(out, 0.0)
    out_ref[...] = _expmap0_proj(out)


def _tc5(x_tan, accyp):
    return pl.pallas_call(
        _tc5_body,
        out_shape=jax.ShapeDtypeStruct((N_NODES, D), jnp.float32),
    )(x_tan, accyp)


_SC_MESH = plsc.VectorSubcoreMesh(core_axis_name="c", subcore_axis_name="s")

EH = N_EDGES // (NC * NS)
HCH = EH // CH


def _sc_hist_body(srcr, dstr, degp, nsp,
                  src_v, dst_v, ew_v, ns_v, zdeg,
                  deg_sh, ns_sh, sem2, sem3):
    c = lax.axis_index("c")
    s = lax.axis_index("s")

    zv = jnp.zeros((16,), jnp.float32)

    def zdrow(j, _):
        zdeg[pl.ds(j * 16, 16)] = zv
        return 0

    lax.fori_loop(0, HIST_PER_TILE // 16, zdrow, 0)
    pltpu.sync_copy(zdeg, deg_sh.at[pl.ds(s * HIST_PER_TILE, HIST_PER_TILE)])
    pltpu.sync_copy(zdeg, ns_sh.at[pl.ds(s * HIST_PER_TILE, HIST_PER_TILE)])

    pltpu.sync_copy(srcr.at[c, s], src_v)
    pltpu.sync_copy(dstr.at[c, s], dst_v)

    def prep(j, _):
        for t in range(CH // 16):
            sl = src_v[j, pl.ds(16 * t, 16)]
            dl = dst_v[j, pl.ds(16 * t, 16)]
            m = sl != dl
            ew_v[j, pl.ds(16 * t, 16)] = jnp.where(m, 1.0, 0.0)
            ns_v[j, pl.ds(16 * t, 16)] = jnp.where(m, 0.0, 1.0)
        return 0

    lax.fori_loop(0, HCH, prep, 0)

    plsc.subcore_barrier()

    def fire(j, _):
        pltpu.async_copy(ew_v.at[j], deg_sh.at[src_v.at[j]], sem2, add=True)
        pltpu.async_copy(ns_v.at[j], ns_sh.at[src_v.at[j]], sem3, add=True)
        return 0

    lax.fori_loop(0, HCH, fire, 0)

    def drain(j, _):
        pltpu.make_async_copy(ew_v.at[0], deg_sh.at[src_v.at[0]],
                              sem2).wait()
        pltpu.make_async_copy(ns_v.at[0], ns_sh.at[src_v.at[0]],
                              sem3).wait()
        return 0

    lax.fori_loop(0, HCH, drain, 0)

    plsc.subcore_barrier()

    hoff = c * HN + s * HIST_PER_TILE
    pltpu.sync_copy(deg_sh.at[pl.ds(s * HIST_PER_TILE, HIST_PER_TILE)],
                    degp.at[pl.ds(hoff, HIST_PER_TILE)])
    pltpu.sync_copy(ns_sh.at[pl.ds(s * HIST_PER_TILE, HIST_PER_TILE)],
                    nsp.at[pl.ds(hoff, HIST_PER_TILE)])


def _sc_segsum_body(table, srcr, dstr, accp,
                    src_v, dst_v, rows0, rows1, zrows,
                    acc_sh, sem0, sem1):
    c = lax.axis_index("c")
    s = lax.axis_index("s")
    base = NR * c

    zv = jnp.zeros((16,), jnp.float32)

    def zrow(j, _):
        for t in range(D // 16):
            zrows[j, pl.ds(16 * t, 16)] = zv
        return 0

    lax.fori_loop(0, ZR, zrow, 0)
    for t in range(STRIPE // ZR):
        pltpu.sync_copy(zrows, acc_sh.at[pl.ds(s * STRIPE + t * ZR, ZR)])

    pltpu.sync_copy(srcr.at[s], src_v)
    pltpu.sync_copy(dstr.at[s], dst_v)

    def prep(j, _):
        for t in range(CH // 16):
            dl = dst_v[j, pl.ds(16 * t, 16)]
            local = dl - base
            inr = (local >= 0) & (local < NR)
            dst_v[j, pl.ds(16 * t, 16)] = jnp.where(
                inr, local, NR + (dl & 7))
        return 0

    lax.fori_loop(0, NCH, prep, 0)

    plsc.subcore_barrier()

    pltpu.async_copy(table.at[src_v.at[0]], rows0, sem0)
    pltpu.async_copy(table.at[src_v.at[1]], rows1, sem1)

    def chunk_pair(jj, _):
        j = 2 * jj
        pltpu.make_async_copy(table.at[src_v.at[j]], rows0, sem0).wait()
        pltpu.sync_copy(rows0, acc_sh.at[dst_v.at[j]], add=True)
        pltpu.async_copy(
            table.at[src_v.at[jnp.minimum(j + 2, NCH - 1)]], rows0, sem0)
        pltpu.make_async_copy(table.at[src_v.at[j + 1]], rows1, sem1).wait()
        pltpu.sync_copy(rows1, acc_sh.at[dst_v.at[j + 1]], add=True)
        pltpu.async_copy(
            table.at[src_v.at[jnp.minimum(j + 3, NCH - 1)]], rows1, sem1)
        return 0

    lax.fori_loop(0, NCH // 2, chunk_pair, 0)

    pltpu.make_async_copy(table.at[src_v.at[0]], rows0, sem0).wait()
    pltpu.make_async_copy(table.at[src_v.at[0]], rows1, sem1).wait()

    plsc.subcore_barrier()

    pltpu.sync_copy(acc_sh.at[pl.ds(s * STRIPE, STRIPE)],
                    accp.at[c, pl.ds(s * STRIPE, STRIPE)])


_sc_hist = pl.kernel(
    _sc_hist_body,
    out_type=(
        jax.ShapeDtypeStruct((NC * HN,), jnp.float32),
        jax.ShapeDtypeStruct((NC * HN,), jnp.float32),
    ),
    mesh=_SC_MESH,
    scratch_types=[
        pltpu.VMEM((HCH, CH), jnp.int32),
        pltpu.VMEM((HCH, CH), jnp.int32),
        pltpu.VMEM((HCH, CH), jnp.float32),
        pltpu.VMEM((HCH, CH), jnp.float32),
        pltpu.VMEM((HIST_PER_TILE,), jnp.float32),
        pltpu.VMEM_SHARED((HN,), jnp.float32),
        pltpu.VMEM_SHARED((HN,), jnp.float32),
        pltpu.SemaphoreType.DMA,
        pltpu.SemaphoreType.DMA,
    ],
)

_sc_segsum = pl.kernel(
    _sc_segsum_body,
    out_type=jax.ShapeDtypeStruct((NC, NR, D), jnp.float32),
    mesh=_SC_MESH,
    scratch_types=[
        pltpu.VMEM((NCH, CH), jnp.int32),
        pltpu.VMEM((NCH, CH), jnp.int32),
        pltpu.VMEM((CH, D), jnp.float32),
        pltpu.VMEM((CH, D), jnp.float32),
        pltpu.VMEM((ZR, D), jnp.float32),
        pltpu.VMEM_SHARED((ACC_R, D), jnp.float32),
        pltpu.SemaphoreType.DMA,
        pltpu.SemaphoreType.DMA,
    ],
)


def kernel(x, edge_index, W, b, Wl, bl):
    srcr = edge_index[0].reshape(NS, NCH, CH)
    dstr = edge_index[1].reshape(NS, NCH, CH)
    srch = edge_index[0].reshape(NC, NS, HCH, CH)
    dsth = edge_index[1].reshape(NC, NS, HCH, CH)

    x_tan = _tc1(x, W, b)

    degp, nsp = _sc_hist(srch, dsth)
    degp = degp.reshape(NC, HN)
    nsp = nsp.reshape(NC, HN)
    deg = (degp[0] + degp[1])[:N_NODES, None]
    ns = (nsp[0] + nsp[1])[:N_NODES, None]
    accxp = _sc_segsum(x_tan, srcr, dstr)
    z, dinv, nself = _tc2(deg, ns, x_tan)
    acczp = _sc_segsum(z, srcr, dstr)
    y2, sel = _tc3(x_tan, z, dinv, nself, acczp)
    accsp = _sc_segsum(y2, srcr, dstr)
    y3 = _tc4(accxp, accsp, x_tan, sel, Wl, bl)
    accyp = _sc_segsum(y3, srcr, dstr)
    return _tc5(x_tan, accyp)
